# baseline (device time: 13285 ns/iter reference)
import os

import jax
import jax.numpy as jnp
from jax import lax
from jax.experimental import pallas as pl
from jax.experimental.pallas import tpu as pltpu

N_DEV = 4
B, SQ, SKV, HQ_LOCAL, DH = 2, 128, 128, 4, 64
D_MODEL = 512

_NO_COMM = os.environ.get("KERNEL_NO_COMM") == "1"


def kernel(x, Wq, K_ext, V_ext, Wo):
    h0 = HQ_LOCAL * lax.axis_index("i")
    K_loc = lax.dynamic_slice_in_dim(K_ext, h0, HQ_LOCAL, axis=2)
    V_loc = lax.dynamic_slice_in_dim(V_ext, h0, HQ_LOCAL, axis=2)

    def body(x_ref, wq_ref, k_ref, v_ref, wo_ref, out_ref,
             acc_ref, send_ref, recv_ref, send_sems, recv_sems):
        my_pos = lax.axis_index("i")
        partner_a = my_pos ^ 1
        partner_b = 3 - my_pos

        if not _NO_COMM:
            barrier_sem = pltpu.get_barrier_semaphore()
            for nbr in (partner_a, partner_b):
                pl.semaphore_signal(
                    barrier_sem, inc=1,
                    device_id=(nbr,), device_id_type=pl.DeviceIdType.MESH,
                )

        CH = 8
        SH = SQ // CH

        def _exchange(stage, b, c, partner):
            sl = pl.ds(c * SH, SH)
            return pltpu.make_async_remote_copy(
                src_ref=send_ref.at[stage, b, sl, :],
                dst_ref=recv_ref.at[stage, b, sl, :],
                send_sem=send_sems.at[stage, b, c],
                recv_sem=recv_sems.at[stage, b, c],
                device_id=(partner,),
                device_id_type=pl.DeviceIdType.MESH,
            )

        _p0 = {0: partner_a, 1: partner_b}
        _p1 = {0: partner_b, 1: partner_a}
        rdma_a = [[_exchange(0, b, c, _p0[c % 2]) for c in range(CH)]
                  for b in range(B)]
        rdma_b = [[_exchange(1, b, c, _p1[c % 2]) for c in range(CH)]
                  for b in range(B)]

        wq = wq_ref[:, :].astype(jnp.bfloat16)
        wo = wo_ref[:, :].astype(jnp.bfloat16)

        for b in range(B):
            xb = x_ref[b, :, :].astype(jnp.bfloat16)
            qb = jnp.dot(xb, wq, preferred_element_type=jnp.float32)
            ctxs = []
            for h in range(HQ_LOCAL):
                qh = qb[:, h * DH:(h + 1) * DH].astype(jnp.bfloat16)
                kh = k_ref[b, :, h, :].astype(jnp.bfloat16)
                vh = v_ref[b, :, h, :].astype(jnp.bfloat16)
                s = jnp.dot(qh, kh.T, preferred_element_type=jnp.float32) * 0.125
                w = jnp.exp(s)
                w = w * (1.0 / jnp.sum(w, axis=-1, keepdims=True))
                ctxs.append(jnp.dot(w.astype(jnp.bfloat16), vh,
                                    preferred_element_type=jnp.float32))
            ctx_b = jnp.concatenate(ctxs, axis=-1)
            pb = jnp.dot(ctx_b.astype(jnp.bfloat16), wo,
                         preferred_element_type=jnp.float32)
            acc_ref[b, :, :] = pb
            send_ref[0, b, :, :] = pb.astype(jnp.bfloat16)
            if not _NO_COMM:
                if b == 0:
                    pl.semaphore_wait(barrier_sem, 2)
                for c in range(CH):
                    rdma_a[b][c].start()

        if _NO_COMM:
            for b in range(B):
                out_ref[b, :, :] = acc_ref[b, :, :].astype(out_ref.dtype)
            return

        for b in range(B):
            for c in range(CH):
                sl = pl.ds(c * SH, SH)
                rdma_a[b][c].wait()
                acc = acc_ref[b, sl, :] + recv_ref[0, b, sl, :].astype(jnp.float32)
                acc_ref[b, sl, :] = acc
                send_ref[1, b, sl, :] = acc.astype(jnp.bfloat16)
                rdma_b[b][c].start()

        for b in range(B):
            for c in range(CH):
                sl = pl.ds(c * SH, SH)
                rdma_b[b][c].wait()
                out_ref[b, sl, :] = (acc_ref[b, sl, :]
                                     + recv_ref[1, b, sl, :].astype(jnp.float32)
                                     ).astype(out_ref.dtype)

    return pl.pallas_call(
        body,
        out_shape=jax.ShapeDtypeStruct((B, SQ, D_MODEL), jnp.bfloat16),
        in_specs=[pl.BlockSpec(memory_space=pltpu.VMEM)] * 5,
        out_specs=pl.BlockSpec(memory_space=pltpu.VMEM),
        scratch_shapes=[
            pltpu.VMEM((B, SQ, D_MODEL), jnp.float32),
            pltpu.VMEM((2, B, SQ, D_MODEL), jnp.bfloat16),
            pltpu.VMEM((2, B, SQ, D_MODEL), jnp.bfloat16),
            pltpu.SemaphoreType.DMA((2, B, 8)),
            pltpu.SemaphoreType.DMA((2, B, 8)),
        ],
        compiler_params=(None if _NO_COMM
                         else pltpu.CompilerParams(collective_id=0)),
    )(x, Wq, K_loc, V_loc, Wo)


# device time: 13182 ns/iter; 1.0078x vs baseline; 1.0078x over previous
import os

import jax
import jax.numpy as jnp
from jax import lax
from jax.experimental import pallas as pl
from jax.experimental.pallas import tpu as pltpu

N_DEV = 4
B, SQ, SKV, HQ_LOCAL, DH = 2, 128, 128, 4, 64
D_MODEL = 512

_NO_COMM = os.environ.get("KERNEL_NO_COMM") == "1"


def kernel(x, Wq, K_ext, V_ext, Wo):
    h0 = HQ_LOCAL * lax.axis_index("i")
    K_loc = lax.dynamic_slice_in_dim(K_ext, h0, HQ_LOCAL, axis=2)
    V_loc = lax.dynamic_slice_in_dim(V_ext, h0, HQ_LOCAL, axis=2)

    def body(x_ref, wq_ref, k_ref, v_ref, wo_ref, out_ref,
             acc_ref, send_ref, recv_ref, send_sems, recv_sems):
        my_pos = lax.axis_index("i")
        partner_a = my_pos ^ 1
        partner_b = 3 - my_pos

        if not _NO_COMM:
            barrier_sem = pltpu.get_barrier_semaphore()
            for nbr in (partner_a, partner_b):
                pl.semaphore_signal(
                    barrier_sem, inc=1,
                    device_id=(nbr,), device_id_type=pl.DeviceIdType.MESH,
                )

        CH = 4
        SH = SQ // CH

        def _exchange(stage, b, c, partner):
            sl = pl.ds(c * SH, SH)
            return pltpu.make_async_remote_copy(
                src_ref=send_ref.at[stage, b, sl, :],
                dst_ref=recv_ref.at[stage, b, sl, :],
                send_sem=send_sems.at[stage, b, c],
                recv_sem=recv_sems.at[stage, b, c],
                device_id=(partner,),
                device_id_type=pl.DeviceIdType.MESH,
            )

        _p0 = {0: partner_a, 1: partner_b}
        _p1 = {0: partner_b, 1: partner_a}
        rdma_a = [[_exchange(0, b, c, _p0[c % 2]) for c in range(CH)]
                  for b in range(B)]
        rdma_b = [[_exchange(1, b, c, _p1[c % 2]) for c in range(CH)]
                  for b in range(B)]

        wq = wq_ref[:, :].astype(jnp.bfloat16)
        wo = wo_ref[:, :].astype(jnp.bfloat16)

        for b in range(B):
            xb = x_ref[b, :, :].astype(jnp.bfloat16)
            qb = jnp.dot(xb, wq, preferred_element_type=jnp.float32)
            ctxs = []
            for h in range(HQ_LOCAL):
                qh = qb[:, h * DH:(h + 1) * DH].astype(jnp.bfloat16)
                kh = k_ref[b, :, h, :].astype(jnp.bfloat16)
                vh = v_ref[b, :, h, :].astype(jnp.bfloat16)
                s = jnp.dot(qh, kh.T, preferred_element_type=jnp.float32) * 0.125
                w = jnp.exp(s)
                w = w * (1.0 / jnp.sum(w, axis=-1, keepdims=True))
                ctxs.append(jnp.dot(w.astype(jnp.bfloat16), vh,
                                    preferred_element_type=jnp.float32))
            ctx_b = jnp.concatenate(ctxs, axis=-1)
            pb = jnp.dot(ctx_b.astype(jnp.bfloat16), wo,
                         preferred_element_type=jnp.float32)
            acc_ref[b, :, :] = pb
            send_ref[0, b, :, :] = pb.astype(jnp.bfloat16)
            if not _NO_COMM:
                if b == 0:
                    pl.semaphore_wait(barrier_sem, 2)
                for c in range(CH):
                    rdma_a[b][c].start()

        if _NO_COMM:
            for b in range(B):
                out_ref[b, :, :] = acc_ref[b, :, :].astype(out_ref.dtype)
            return

        for b in range(B):
            for c in range(CH):
                sl = pl.ds(c * SH, SH)
                rdma_a[b][c].wait()
                acc = acc_ref[b, sl, :] + recv_ref[0, b, sl, :].astype(jnp.float32)
                acc_ref[b, sl, :] = acc
                send_ref[1, b, sl, :] = acc.astype(jnp.bfloat16)
                rdma_b[b][c].start()

        for b in range(B):
            for c in range(CH):
                sl = pl.ds(c * SH, SH)
                rdma_b[b][c].wait()
                out_ref[b, sl, :] = (acc_ref[b, sl, :]
                                     + recv_ref[1, b, sl, :].astype(jnp.float32)
                                     ).astype(out_ref.dtype)

    return pl.pallas_call(
        body,
        out_shape=jax.ShapeDtypeStruct((B, SQ, D_MODEL), jnp.bfloat16),
        in_specs=[pl.BlockSpec(memory_space=pltpu.VMEM)] * 5,
        out_specs=pl.BlockSpec(memory_space=pltpu.VMEM),
        scratch_shapes=[
            pltpu.VMEM((B, SQ, D_MODEL), jnp.float32),
            pltpu.VMEM((2, B, SQ, D_MODEL), jnp.bfloat16),
            pltpu.VMEM((2, B, SQ, D_MODEL), jnp.bfloat16),
            pltpu.SemaphoreType.DMA((2, B, 4)),
            pltpu.SemaphoreType.DMA((2, B, 4)),
        ],
        compiler_params=(None if _NO_COMM
                         else pltpu.CompilerParams(collective_id=0)),
    )(x, Wq, K_loc, V_loc, Wo)


# device time: 13017 ns/iter; 1.0206x vs baseline; 1.0127x over previous
import os

import jax
import jax.numpy as jnp
from jax import lax
from jax.experimental import pallas as pl
from jax.experimental.pallas import tpu as pltpu

N_DEV = 4
B, SQ, SKV, HQ_LOCAL, DH = 2, 128, 128, 4, 64
D_MODEL = 512

_NO_COMM = os.environ.get("KERNEL_NO_COMM") == "1"


def kernel(x, Wq, K_ext, V_ext, Wo):
    h0 = HQ_LOCAL * lax.axis_index("i")
    K_loc = lax.dynamic_slice_in_dim(K_ext, h0, HQ_LOCAL, axis=2).reshape(
        B, SKV, HQ_LOCAL * DH)
    V_loc = lax.dynamic_slice_in_dim(V_ext, h0, HQ_LOCAL, axis=2).reshape(
        B, SKV, HQ_LOCAL * DH)

    def body(x_ref, wq_ref, k_ref, v_ref, wo_ref, out_ref,
             acc_ref, send_ref, recv_ref, send_sems, recv_sems):
        my_pos = lax.axis_index("i")
        partner_a = my_pos ^ 1
        partner_b = 3 - my_pos

        if not _NO_COMM:
            barrier_sem = pltpu.get_barrier_semaphore()
            for nbr in (partner_a, partner_b):
                pl.semaphore_signal(
                    barrier_sem, inc=1,
                    device_id=(nbr,), device_id_type=pl.DeviceIdType.MESH,
                )

        CH = 4
        SH = SQ // CH

        def _exchange(stage, b, c, partner):
            sl = pl.ds(c * SH, SH)
            return pltpu.make_async_remote_copy(
                src_ref=send_ref.at[stage, b, sl, :],
                dst_ref=recv_ref.at[stage, b, sl, :],
                send_sem=send_sems.at[stage, b, c],
                recv_sem=recv_sems.at[stage, b, c],
                device_id=(partner,),
                device_id_type=pl.DeviceIdType.MESH,
            )

        _p0 = {0: partner_a, 1: partner_b}
        _p1 = {0: partner_b, 1: partner_a}
        rdma_a = [[_exchange(0, b, c, _p0[c % 2]) for c in range(CH)]
                  for b in range(B)]
        rdma_b = [[_exchange(1, b, c, _p1[c % 2]) for c in range(CH)]
                  for b in range(B)]

        wq = wq_ref[:, :].astype(jnp.bfloat16)
        wo = wo_ref[:, :].astype(jnp.bfloat16)

        for b in range(B):
            xb = x_ref[b, :, :].astype(jnp.bfloat16)
            qb = jnp.dot(xb, wq, preferred_element_type=jnp.float32)
            ctxs = []
            for h in range(HQ_LOCAL):
                qh = qb[:, h * DH:(h + 1) * DH].astype(jnp.bfloat16)
                kh = k_ref[b, :, h * DH:(h + 1) * DH].astype(jnp.bfloat16)
                vh = v_ref[b, :, h * DH:(h + 1) * DH].astype(jnp.bfloat16)
                s = jnp.dot(qh, kh.T, preferred_element_type=jnp.float32) * 0.125
                w = jnp.exp(s)
                w = w * (1.0 / jnp.sum(w, axis=-1, keepdims=True))
                ctxs.append(jnp.dot(w.astype(jnp.bfloat16), vh,
                                    preferred_element_type=jnp.float32))
            ctx_b = jnp.concatenate(ctxs, axis=-1)
            pb = jnp.dot(ctx_b.astype(jnp.bfloat16), wo,
                         preferred_element_type=jnp.float32)
            acc_ref[b, :, :] = pb
            send_ref[0, b, :, :] = pb.astype(jnp.bfloat16)
            if not _NO_COMM:
                if b == 0:
                    pl.semaphore_wait(barrier_sem, 2)
                for c in range(CH):
                    rdma_a[b][c].start()

        if _NO_COMM:
            for b in range(B):
                out_ref[b, :, :] = acc_ref[b, :, :].astype(out_ref.dtype)
            return

        for b in range(B):
            for c in range(CH):
                sl = pl.ds(c * SH, SH)
                rdma_a[b][c].wait()
                acc = acc_ref[b, sl, :] + recv_ref[0, b, sl, :].astype(jnp.float32)
                acc_ref[b, sl, :] = acc
                send_ref[1, b, sl, :] = acc.astype(jnp.bfloat16)
                rdma_b[b][c].start()

        for b in range(B):
            for c in range(CH):
                sl = pl.ds(c * SH, SH)
                rdma_b[b][c].wait()
                out_ref[b, sl, :] = (acc_ref[b, sl, :]
                                     + recv_ref[1, b, sl, :].astype(jnp.float32)
                                     ).astype(out_ref.dtype)

    return pl.pallas_call(
        body,
        out_shape=jax.ShapeDtypeStruct((B, SQ, D_MODEL), jnp.bfloat16),
        in_specs=[pl.BlockSpec(memory_space=pltpu.VMEM)] * 5,
        out_specs=pl.BlockSpec(memory_space=pltpu.VMEM),
        scratch_shapes=[
            pltpu.VMEM((B, SQ, D_MODEL), jnp.float32),
            pltpu.VMEM((2, B, SQ, D_MODEL), jnp.bfloat16),
            pltpu.VMEM((2, B, SQ, D_MODEL), jnp.bfloat16),
            pltpu.SemaphoreType.DMA((2, B, 4)),
            pltpu.SemaphoreType.DMA((2, B, 4)),
        ],
        compiler_params=(None if _NO_COMM
                         else pltpu.CompilerParams(collective_id=0)),
    )(x, Wq, K_loc, V_loc, Wo)
